# baseline (speedup 1.0000x reference)
import functools

import jax
import jax.numpy as jnp
import numpy as np
from jax import lax
from jax.experimental import pallas as pl
from jax.experimental.pallas import tpu as pltpu
from jax.experimental.pallas import tpu_sc as plsc

T, B, C = 196, 256, 768
RATIO = 0.75
REMAIN = int(T * (1 - RATIO))
DROP = T - REMAIN
ROWS = T * B
ROWS1 = REMAIN * B
ROWS2 = DROP * B
NC, NS, L = 2, 16, 16
NW = NC * NS
GPW = ROWS2 // NW
GCH = 24
NGCH = GPW // GCH
NB = 7
KLA = 4
NROUND = NGCH // NB
GBLK = 74
GPAD = GBLK * L
FPAD = ROWS1 + NW * GPW + GPAD
FLEN = ((FPAD + 127) // 128) * 128

_mesh = plsc.VectorSubcoreMesh(
    core_axis_name="c", subcore_axis_name="s", num_cores=NC, num_subcores=NS
)


@functools.partial(
    pl.kernel,
    mesh=_mesh,
    out_type=jax.ShapeDtypeStruct((ROWS2, C), jnp.float32),
    scratch_types=[
        pltpu.VMEM((GPAD,), jnp.int32),
        [pltpu.VMEM((GCH, C), jnp.float32)] * NB,
        [pltpu.SemaphoreType.DMA] * NB,
        [pltpu.SemaphoreType.DMA] * NB,
    ],
)
def _shuffle_sc(fwd_flat_hbm, table_hbm, out2_hbm, idxg_v, rows, gsem, ssem):
    w = lax.axis_index("s") * NC + lax.axis_index("c")
    lane = lax.iota(jnp.int32, L)

    gbase = ROWS1 + w * GPW
    pltpu.sync_copy(fwd_flat_hbm.at[pl.ds(gbase, GPAD)], idxg_v)

    def gbuild(j, carry):
        off = pl.multiple_of(j * L, 8)
        f = idxg_v[pl.ds(off, L)]
        rv = (gbase + j * L) + lane
        idxg_v[pl.ds(off, L)] = f * B + lax.rem(rv, B)
        return carry

    lax.fori_loop(0, GBLK, gbuild, 0)

    def _gath(h, s):
        iref = idxg_v.at[pl.ds(h * GCH, GCH)]
        return pltpu.make_async_copy(table_hbm.at[iref], rows[s], gsem[s])

    def _stor(h, s):
        dst = out2_hbm.at[pl.ds(w * GPW + h * GCH, GCH), :]
        return pltpu.make_async_copy(rows[s], dst, ssem[s])

    for h in range(KLA):
        _gath(h, h).start()

    def round_body(r, carry):
        for s in range(NB):
            h = r * NB + s
            _gath(h, s).wait()
            _stor(h, s).start()
            s2 = (s + KLA) % NB
            h2 = h + KLA

            @pl.when(h2 < NGCH)
            def _():
                @pl.when(h2 >= NB)
                def _():
                    _stor(h2 - NB, s2).wait()
                _gath(h2, s2).start()

        return carry

    lax.fori_loop(0, NROUND, round_body, 0)
    for s in range(NB):
        _stor(NGCH - NB + s, s).wait()


BG = 16


def _tc_body(fref, pref, oref):
    for k in range(BG):
        f = fref[0, k, :]
        x = pref[:, k, :]
        oh = (f[:REMAIN, None] ==
              lax.broadcasted_iota(jnp.int32, (REMAIN, T), 1))
        oref[:, k, :] = jnp.dot(oh.astype(jnp.float32), x,
                                preferred_element_type=jnp.float32)


_tc_shuffle = pl.pallas_call(
    _tc_body,
    grid=(B // BG,),
    in_specs=[
        pl.BlockSpec((1, BG, T), lambda b: (b, 0, 0)),
        pl.BlockSpec((T, BG, C), lambda b: (0, b, 0)),
    ],
    out_specs=pl.BlockSpec((REMAIN, BG, C), lambda b: (0, b, 0)),
    out_shape=jax.ShapeDtypeStruct((REMAIN, B, C), jnp.float32),
)


def _forward_indexes():
    keys = jax.random.split(jax.random.key(1), B)
    fwd = jax.vmap(lambda k: jax.random.permutation(k, T))(keys).T
    return fwd.astype(jnp.int32)


def _index_consts():
    try:
        with jax.default_device(jax.devices("cpu")[0]):
            fwd = np.asarray(_forward_indexes())
        bwd = np.argsort(fwd, axis=0).astype(np.int32)
        return fwd, bwd
    except Exception:
        return None, None


_FWD_CONST, _BWD_CONST = _index_consts()


def kernel(patches):
    if _FWD_CONST is not None:
        fwd = jnp.asarray(_FWD_CONST)
        bwd = jnp.asarray(_BWD_CONST)
    else:
        fwd = _forward_indexes()
        bwd = jnp.argsort(fwd, axis=0).astype(jnp.int32)
    table = patches.reshape(ROWS, C)
    fwd_flat = jnp.pad(fwd.reshape(ROWS), (0, FLEN - ROWS))
    out2 = _shuffle_sc(fwd_flat, table)
    fwdt = fwd.T.reshape(B // BG, BG, T)
    patches_1 = _tc_shuffle(fwdt, patches)
    patches_2 = out2.reshape(DROP, B, C)
    return (patches_1, patches_2,
            fwd.astype(jnp.int64), bwd.astype(jnp.int64))
